# baseline (speedup 1.0000x reference)
import functools

import jax
import jax.numpy as jnp
from jax import lax
from jax.experimental import pallas as pl
from jax.experimental.pallas import tpu as pltpu
from jax.experimental.pallas import tpu_sc as plsc

N = 10000
T = 8
F_DIM = 128
H = 128
E = 320000
ALPHA = 0.5

NC = 2
NS = 16
EPT = E // NS
CHB = 80
NCH = EPT // CHB
PAIRS = T * 2
PPC = PAIRS // NC
ROWS_PT = N // NS

_mesh = plsc.VectorSubcoreMesh(core_axis_name="c", subcore_axis_name="s")


def _deg_body(ei_hbm, ones_hbm, zeros_hbm, din_hbm, dout_hbm, idx_v, ones_v,
              bounce_v, sem, hist_sp):
    c = lax.axis_index("c")
    s = lax.axis_index("s")

    pltpu.sync_copy(ones_hbm, ones_v)
    pltpu.sync_copy(zeros_hbm, bounce_v)
    row = c * PPC + s // 2
    col = (s % 2) * (N // 2)
    pltpu.sync_copy(bounce_v, hist_sp.at[row, pl.ds(col, N // 2)])
    plsc.subcore_barrier()
    for tt in range(T // NC):
        for dd in range(2):
            t = c * (T // NC) + tt
            tp = t * 2 + dd
            pltpu.sync_copy(ei_hbm.at[t, 1 - dd, pl.ds(s * EPT, EPT)], idx_v)

            @pl.loop(0, NCH)
            def _(ch):
                pltpu.sync_copy(ones_v,
                                hist_sp.at[tp].at[idx_v.at[pl.ds(ch * CHB, CHB)]],
                                add=True)

    plsc.subcore_barrier()
    pltpu.sync_copy(hist_sp.at[row, pl.ds(col, N // 2)], bounce_v)
    t_row = row // 2

    @pl.when(row % 2 == 0)
    def _():
        pltpu.sync_copy(bounce_v, din_hbm.at[t_row, pl.ds(col, N // 2)])

    @pl.when(row % 2 == 1)
    def _():
        pltpu.sync_copy(bounce_v, dout_hbm.at[t_row, pl.ds(col, N // 2)])


_deg_call = pl.kernel(
    _deg_body,
    out_type=[jax.ShapeDtypeStruct((T, N), jnp.float32),
              jax.ShapeDtypeStruct((T, N), jnp.float32)],
    mesh=_mesh,
    compiler_params=pltpu.CompilerParams(use_tc_tiling_on_sc=False),
    scratch_types=[
        pltpu.VMEM((EPT,), jnp.int32),
        pltpu.VMEM((CHB,), jnp.float32),
        pltpu.VMEM((N // 2,), jnp.float32),
        pltpu.SemaphoreType.DMA,
        pltpu.VMEM_SHARED((PAIRS, N), jnp.float32),
    ],
)


NBLK = 5
NCHB = NCH // NBLK


def _agg_body(table_hbm, ei_hbm, zrows_hbm, out_hbm,
              gidx_v, sidx_v, rows_a, rows_b, sem_ga, sem_gb, sem_sa, sem_sb,
              acc_sp):
    c = lax.axis_index("c")
    s = lax.axis_index("s")


    def wait_g(buf, sem):
        pltpu.make_async_copy(table_hbm.at[0, pl.ds(0, CHB)], buf, sem).wait()

    def sc(ch, buf, sem):
        pltpu.async_copy(buf, acc_sp.at[sidx_v.at[pl.ds(ch * CHB, CHB)]], sem,
                         add=True)

    def wait_s(buf, sem):
        pltpu.make_async_copy(buf, acc_sp.at[sidx_v.at[pl.ds(0, CHB)]], sem).wait()

    for tt in range(T // NC):
        for dd in range(2):
            t = c * (T // NC) + tt
            tp = t * 2 + dd
            tbl = table_hbm.at[tp]
            pltpu.sync_copy(zrows_hbm, acc_sp.at[pl.ds(s * ROWS_PT, ROWS_PT)])
            plsc.subcore_barrier()
            for hh in range(NBLK):
                pltpu.sync_copy(
                    ei_hbm.at[t, 1 - dd, pl.ds(s * EPT + hh * NCHB * CHB, NCHB * CHB)],
                    gidx_v)
                pltpu.sync_copy(
                    ei_hbm.at[t, dd, pl.ds(s * EPT + hh * NCHB * CHB, NCHB * CHB)],
                    sidx_v)

                def gg(ch, buf, sem):
                    pltpu.async_copy(tbl.at[gidx_v.at[pl.ds(ch * CHB, CHB)]], buf, sem)

                gg(0, rows_a, sem_ga)
                gg(1, rows_b, sem_gb)
                wait_g(rows_a, sem_ga)
                sc(0, rows_a, sem_sa)

                @pl.loop(0, NCHB // 2 - 1)
                def _(i):
                    ch = 2 * i
                    wait_s(rows_a, sem_sa)
                    gg(ch + 2, rows_a, sem_ga)
                    wait_g(rows_b, sem_gb)
                    sc(ch + 1, rows_b, sem_sb)
                    wait_s(rows_b, sem_sb)
                    gg(ch + 3, rows_b, sem_gb)
                    wait_g(rows_a, sem_ga)
                    sc(ch + 2, rows_a, sem_sa)

                wait_s(rows_a, sem_sa)
                wait_g(rows_b, sem_gb)
                sc(NCHB - 1, rows_b, sem_sb)
                wait_s(rows_b, sem_sb)

            plsc.subcore_barrier()
            pltpu.sync_copy(acc_sp.at[pl.ds(s * ROWS_PT, ROWS_PT)],
                            out_hbm.at[t, dd, pl.ds(s * ROWS_PT, ROWS_PT)])


_agg_call = pl.kernel(
    _agg_body,
    out_type=jax.ShapeDtypeStruct((T, 2, N, F_DIM), jnp.float32),
    mesh=_mesh,
    compiler_params=pltpu.CompilerParams(use_tc_tiling_on_sc=False),
    scratch_types=[
        pltpu.VMEM((NCHB * CHB,), jnp.int32),
        pltpu.VMEM((NCHB * CHB,), jnp.int32),
        pltpu.VMEM((CHB, F_DIM), jnp.float32),
        pltpu.VMEM((CHB, F_DIM), jnp.float32),
        pltpu.SemaphoreType.DMA,
        pltpu.SemaphoreType.DMA,
        pltpu.SemaphoreType.DMA,
        pltpu.SemaphoreType.DMA,
        pltpu.VMEM_SHARED((N, F_DIM), jnp.float32),
    ],
)


BN = 2000
NB = N // BN


def _inv_sqrt(d):
    return jnp.where(d > 0, lax.rsqrt(jnp.maximum(d, 1e-12)), 0.0)


def _scale_body(x_ref, di_ref, do_ref, out_ref):
    x = x_ref[0]
    inv_i = _inv_sqrt(di_ref[0, 0])
    inv_o = _inv_sqrt(do_ref[0, 0])
    out_ref[0, 0] = x * inv_i
    out_ref[0, 1] = x * inv_o


_scale_call = pl.pallas_call(
    _scale_body,
    grid=(T, NB),
    in_specs=[
        pl.BlockSpec((1, BN, F_DIM), lambda t, n: (t, n, 0)),
        pl.BlockSpec((1, 1, BN, 1), lambda t, n: (t, n, 0, 0)),
        pl.BlockSpec((1, 1, BN, 1), lambda t, n: (t, n, 0, 0)),
    ],
    out_specs=pl.BlockSpec((1, 2, BN, F_DIM), lambda t, n: (t, 0, n, 0)),
    out_shape=jax.ShapeDtypeStruct((T, 2, N, F_DIM), jnp.float32),
)


def _combine_body(u_ref, di_ref, do_ref, ws_ref, bs_ref, wd_ref, bd_ref,
                  out_ref, *, relu_and_scale):
    inv_i = _inv_sqrt(di_ref[0, 0])
    inv_o = _inv_sqrt(do_ref[0, 0])
    agg_fwd = u_ref[0, 0] * inv_o
    agg_rev = u_ref[0, 1] * inv_i
    dn = (((1,), (1,)), ((), ()))
    h = (ALPHA * (lax.dot_general(agg_fwd, ws_ref[...], dn,
                                  preferred_element_type=jnp.float32)
                  + bs_ref[...])
         + (1.0 - ALPHA) * (lax.dot_general(agg_rev, wd_ref[...], dn,
                                            preferred_element_type=jnp.float32)
                            + bd_ref[...]))
    if relu_and_scale:
        h = jnp.maximum(h, 0.0)
        out_ref[0, 0] = h * inv_i
        out_ref[0, 1] = h * inv_o
    else:
        out_ref[0] = h


def _make_combine(relu_and_scale):
    if relu_and_scale:
        out_specs = pl.BlockSpec((1, 2, BN, F_DIM), lambda t, n: (t, 0, n, 0))
        out_shape = jax.ShapeDtypeStruct((T, 2, N, F_DIM), jnp.float32)
    else:
        out_specs = pl.BlockSpec((1, BN, F_DIM), lambda t, n: (t, n, 0))
        out_shape = jax.ShapeDtypeStruct((T, N, F_DIM), jnp.float32)
    return pl.pallas_call(
        functools.partial(_combine_body, relu_and_scale=relu_and_scale),
        grid=(T, NB),
        in_specs=[
            pl.BlockSpec((1, 2, BN, F_DIM), lambda t, n: (t, 0, n, 0)),
            pl.BlockSpec((1, 1, BN, 1), lambda t, n: (t, n, 0, 0)),
            pl.BlockSpec((1, 1, BN, 1), lambda t, n: (t, n, 0, 0)),
            pl.BlockSpec((H, F_DIM), lambda t, n: (0, 0)),
            pl.BlockSpec((1, H), lambda t, n: (0, 0)),
            pl.BlockSpec((H, F_DIM), lambda t, n: (0, 0)),
            pl.BlockSpec((1, H), lambda t, n: (0, 0)),
        ],
        out_specs=out_specs,
        out_shape=out_shape,
    )


_combine1_call = _make_combine(True)
_combine2_call = _make_combine(False)

BL = 2000
NBL = N // BL


def _lstm_body(seq_ref, wih_ref, whh_ref, bih_ref, bhh_ref, wp_ref, bp_ref,
               out_ref):
    dn = (((1,), (1,)), ((), ()))
    b = bih_ref[...] + bhh_ref[...]
    h = jnp.zeros((BL, H), jnp.float32)
    c = jnp.zeros((BL, H), jnp.float32)
    for t in range(T):
        xt = seq_ref[t]
        gates = (lax.dot_general(xt, wih_ref[...], dn,
                                 preferred_element_type=jnp.float32)
                 + lax.dot_general(h, whh_ref[...], dn,
                                   preferred_element_type=jnp.float32)
                 + b)
        i = jax.nn.sigmoid(gates[:, 0:H])
        f = jax.nn.sigmoid(gates[:, H:2 * H])
        g = jnp.tanh(gates[:, 2 * H:3 * H])
        o = jax.nn.sigmoid(gates[:, 3 * H:4 * H])
        c = f * c + i * g
        h = o * jnp.tanh(c)
    out_ref[...] = (lax.dot_general(h, wp_ref[...], dn,
                                    preferred_element_type=jnp.float32)
                    + bp_ref[...])


_lstm_call = pl.pallas_call(
    _lstm_body,
    grid=(NBL,),
    in_specs=[
        pl.BlockSpec((T, BL, H), lambda n: (0, n, 0)),
        pl.BlockSpec((4 * H, H), lambda n: (0, 0)),
        pl.BlockSpec((4 * H, H), lambda n: (0, 0)),
        pl.BlockSpec((1, 4 * H), lambda n: (0, 0)),
        pl.BlockSpec((1, 4 * H), lambda n: (0, 0)),
        pl.BlockSpec((F_DIM, H), lambda n: (0, 0)),
        pl.BlockSpec((1, F_DIM), lambda n: (0, 0)),
    ],
    out_specs=pl.BlockSpec((BL, F_DIM), lambda n: (n, 0)),
    out_shape=jax.ShapeDtypeStruct((N, F_DIM), jnp.float32),
)


def kernel(x_sequence, W_s1, b_s1, W_d1, b_d1, W_s2, b_s2, W_d2, b_d2,
           W_ih, W_hh, b_ih, b_hh, W_p, b_p, edge_index_sequence):
    ei = edge_index_sequence

    ones_chb = jnp.ones((CHB,), jnp.float32)
    zeros_hist = jnp.zeros((N // 2,), jnp.float32)
    zeros_rows = jnp.zeros((ROWS_PT, F_DIM), jnp.float32)

    deg_in, deg_out = _deg_call(ei, ones_chb, zeros_hist)
    deg_in = deg_in.reshape(T, NB, BN, 1)
    deg_out = deg_out.reshape(T, NB, BN, 1)

    xcat = _scale_call(x_sequence, deg_in, deg_out)
    u1 = _agg_call(xcat.reshape(PAIRS, N, F_DIM), ei, zeros_rows)
    h1cat = _combine1_call(u1, deg_in, deg_out, W_s1, b_s1.reshape(1, H),
                           W_d1, b_d1.reshape(1, H))
    u2 = _agg_call(h1cat.reshape(PAIRS, N, F_DIM), ei, zeros_rows)
    h2 = _combine2_call(u2, deg_in, deg_out, W_s2, b_s2.reshape(1, H),
                        W_d2, b_d2.reshape(1, H))
    out = _lstm_call(h2, W_ih, W_hh, b_ih.reshape(1, 4 * H),
                     b_hh.reshape(1, 4 * H), W_p, b_p.reshape(1, F_DIM))
    return out


# speedup vs baseline: 1.1164x; 1.1164x over previous
import functools

import jax
import jax.numpy as jnp
from jax import lax
from jax.experimental import pallas as pl
from jax.experimental.pallas import tpu as pltpu
from jax.experimental.pallas import tpu_sc as plsc

N = 10000
T = 8
F_DIM = 128
H = 128
E = 320000
ALPHA = 0.5

NC = 2
NS = 16
EPT = E // NS
CHB = 160
NCH = EPT // CHB
PAIRS = T * 2
PPC = PAIRS // NC
ROWS_PT = N // NS

_mesh = plsc.VectorSubcoreMesh(core_axis_name="c", subcore_axis_name="s")


def _deg_body(ei_hbm, ones_hbm, zeros_hbm, din_hbm, dout_hbm, idx_v, ones_v,
              bounce_v, sem, hist_sp):
    c = lax.axis_index("c")
    s = lax.axis_index("s")

    pltpu.sync_copy(ones_hbm, ones_v)
    pltpu.sync_copy(zeros_hbm, bounce_v)
    row = c * PPC + s // 2
    col = (s % 2) * (N // 2)
    pltpu.sync_copy(bounce_v, hist_sp.at[row, pl.ds(col, N // 2)])
    plsc.subcore_barrier()
    for tt in range(T // NC):
        for dd in range(2):
            t = c * (T // NC) + tt
            tp = t * 2 + dd
            pltpu.sync_copy(ei_hbm.at[t, 1 - dd, pl.ds(s * EPT, EPT)], idx_v)

            @pl.loop(0, NCH)
            def _(ch):
                pltpu.sync_copy(ones_v,
                                hist_sp.at[tp].at[idx_v.at[pl.ds(ch * CHB, CHB)]],
                                add=True)

    plsc.subcore_barrier()
    pltpu.sync_copy(hist_sp.at[row, pl.ds(col, N // 2)], bounce_v)
    t_row = row // 2

    @pl.when(row % 2 == 0)
    def _():
        pltpu.sync_copy(bounce_v, din_hbm.at[t_row, pl.ds(col, N // 2)])

    @pl.when(row % 2 == 1)
    def _():
        pltpu.sync_copy(bounce_v, dout_hbm.at[t_row, pl.ds(col, N // 2)])


_deg_call = pl.kernel(
    _deg_body,
    out_type=[jax.ShapeDtypeStruct((T, N), jnp.float32),
              jax.ShapeDtypeStruct((T, N), jnp.float32)],
    mesh=_mesh,
    compiler_params=pltpu.CompilerParams(use_tc_tiling_on_sc=False),
    scratch_types=[
        pltpu.VMEM((EPT,), jnp.int32),
        pltpu.VMEM((CHB,), jnp.float32),
        pltpu.VMEM((N // 2,), jnp.float32),
        pltpu.SemaphoreType.DMA,
        pltpu.VMEM_SHARED((PAIRS, N), jnp.float32),
    ],
)


NBLK = 5
NCHB = NCH // NBLK


def _agg_body(table_hbm, ei_hbm, zrows_hbm, out_hbm,
              gidx_v, sidx_v, rows_a, rows_b, sem_ga, sem_gb, sem_sa, sem_sb,
              acc_sp):
    c = lax.axis_index("c")
    s = lax.axis_index("s")


    def wait_g(buf, sem):
        pltpu.make_async_copy(table_hbm.at[0, pl.ds(0, CHB)], buf, sem).wait()

    def sc(ch, buf, sem):
        pltpu.async_copy(buf, acc_sp.at[sidx_v.at[pl.ds(ch * CHB, CHB)]], sem,
                         add=True)

    def wait_s(buf, sem):
        pltpu.make_async_copy(buf, acc_sp.at[sidx_v.at[pl.ds(0, CHB)]], sem).wait()

    for tt in range(T // NC):
        for dd in range(2):
            t = c * (T // NC) + tt
            tp = t * 2 + dd
            tbl = table_hbm.at[tp]
            pltpu.sync_copy(zrows_hbm, acc_sp.at[pl.ds(s * ROWS_PT, ROWS_PT)])
            plsc.subcore_barrier()
            for hh in range(NBLK):
                pltpu.sync_copy(
                    ei_hbm.at[t, 1 - dd, pl.ds(s * EPT + hh * NCHB * CHB, NCHB * CHB)],
                    gidx_v)
                pltpu.sync_copy(
                    ei_hbm.at[t, dd, pl.ds(s * EPT + hh * NCHB * CHB, NCHB * CHB)],
                    sidx_v)

                def gg(ch, buf, sem):
                    pltpu.async_copy(tbl.at[gidx_v.at[pl.ds(ch * CHB, CHB)]], buf, sem)

                gg(0, rows_a, sem_ga)
                gg(1, rows_b, sem_gb)
                wait_g(rows_a, sem_ga)
                sc(0, rows_a, sem_sa)

                @pl.loop(0, NCHB // 2 - 1)
                def _(i):
                    ch = 2 * i
                    wait_s(rows_a, sem_sa)
                    gg(ch + 2, rows_a, sem_ga)
                    wait_g(rows_b, sem_gb)
                    sc(ch + 1, rows_b, sem_sb)
                    wait_s(rows_b, sem_sb)
                    gg(ch + 3, rows_b, sem_gb)
                    wait_g(rows_a, sem_ga)
                    sc(ch + 2, rows_a, sem_sa)

                wait_s(rows_a, sem_sa)
                gg(NCHB - 1, rows_a, sem_ga)
                wait_g(rows_b, sem_gb)
                sc(NCHB - 2, rows_b, sem_sb)
                wait_g(rows_a, sem_ga)
                sc(NCHB - 1, rows_a, sem_sa)
                wait_s(rows_b, sem_sb)
                wait_s(rows_a, sem_sa)

            plsc.subcore_barrier()
            pltpu.sync_copy(acc_sp.at[pl.ds(s * ROWS_PT, ROWS_PT)],
                            out_hbm.at[t, dd, pl.ds(s * ROWS_PT, ROWS_PT)])


_agg_call = pl.kernel(
    _agg_body,
    out_type=jax.ShapeDtypeStruct((T, 2, N, F_DIM), jnp.float32),
    mesh=_mesh,
    compiler_params=pltpu.CompilerParams(use_tc_tiling_on_sc=False),
    scratch_types=[
        pltpu.VMEM((NCHB * CHB,), jnp.int32),
        pltpu.VMEM((NCHB * CHB,), jnp.int32),
        pltpu.VMEM((CHB, F_DIM), jnp.float32),
        pltpu.VMEM((CHB, F_DIM), jnp.float32),
        pltpu.SemaphoreType.DMA,
        pltpu.SemaphoreType.DMA,
        pltpu.SemaphoreType.DMA,
        pltpu.SemaphoreType.DMA,
        pltpu.VMEM_SHARED((N, F_DIM), jnp.float32),
    ],
)


BN = 2000
NB = N // BN


def _inv_sqrt(d):
    return jnp.where(d > 0, lax.rsqrt(jnp.maximum(d, 1e-12)), 0.0)


def _scale_body(x_ref, di_ref, do_ref, out_ref):
    x = x_ref[0]
    inv_i = _inv_sqrt(di_ref[0, 0])
    inv_o = _inv_sqrt(do_ref[0, 0])
    out_ref[0, 0] = x * inv_i
    out_ref[0, 1] = x * inv_o


_scale_call = pl.pallas_call(
    _scale_body,
    grid=(T, NB),
    in_specs=[
        pl.BlockSpec((1, BN, F_DIM), lambda t, n: (t, n, 0)),
        pl.BlockSpec((1, 1, BN, 1), lambda t, n: (t, n, 0, 0)),
        pl.BlockSpec((1, 1, BN, 1), lambda t, n: (t, n, 0, 0)),
    ],
    out_specs=pl.BlockSpec((1, 2, BN, F_DIM), lambda t, n: (t, 0, n, 0)),
    out_shape=jax.ShapeDtypeStruct((T, 2, N, F_DIM), jnp.float32),
)


def _combine_body(u_ref, di_ref, do_ref, ws_ref, bs_ref, wd_ref, bd_ref,
                  out_ref, *, relu_and_scale):
    inv_i = _inv_sqrt(di_ref[0, 0])
    inv_o = _inv_sqrt(do_ref[0, 0])
    agg_fwd = u_ref[0, 0] * inv_o
    agg_rev = u_ref[0, 1] * inv_i
    dn = (((1,), (1,)), ((), ()))
    h = (ALPHA * (lax.dot_general(agg_fwd, ws_ref[...], dn,
                                  preferred_element_type=jnp.float32)
                  + bs_ref[...])
         + (1.0 - ALPHA) * (lax.dot_general(agg_rev, wd_ref[...], dn,
                                            preferred_element_type=jnp.float32)
                            + bd_ref[...]))
    if relu_and_scale:
        h = jnp.maximum(h, 0.0)
        out_ref[0, 0] = h * inv_i
        out_ref[0, 1] = h * inv_o
    else:
        out_ref[0] = h


def _make_combine(relu_and_scale):
    if relu_and_scale:
        out_specs = pl.BlockSpec((1, 2, BN, F_DIM), lambda t, n: (t, 0, n, 0))
        out_shape = jax.ShapeDtypeStruct((T, 2, N, F_DIM), jnp.float32)
    else:
        out_specs = pl.BlockSpec((1, BN, F_DIM), lambda t, n: (t, n, 0))
        out_shape = jax.ShapeDtypeStruct((T, N, F_DIM), jnp.float32)
    return pl.pallas_call(
        functools.partial(_combine_body, relu_and_scale=relu_and_scale),
        grid=(T, NB),
        in_specs=[
            pl.BlockSpec((1, 2, BN, F_DIM), lambda t, n: (t, 0, n, 0)),
            pl.BlockSpec((1, 1, BN, 1), lambda t, n: (t, n, 0, 0)),
            pl.BlockSpec((1, 1, BN, 1), lambda t, n: (t, n, 0, 0)),
            pl.BlockSpec((H, F_DIM), lambda t, n: (0, 0)),
            pl.BlockSpec((1, H), lambda t, n: (0, 0)),
            pl.BlockSpec((H, F_DIM), lambda t, n: (0, 0)),
            pl.BlockSpec((1, H), lambda t, n: (0, 0)),
        ],
        out_specs=out_specs,
        out_shape=out_shape,
    )


_combine1_call = _make_combine(True)
_combine2_call = _make_combine(False)

BL = 2000
NBL = N // BL


def _lstm_body(seq_ref, wih_ref, whh_ref, bih_ref, bhh_ref, wp_ref, bp_ref,
               out_ref):
    dn = (((1,), (1,)), ((), ()))
    b = bih_ref[...] + bhh_ref[...]
    h = jnp.zeros((BL, H), jnp.float32)
    c = jnp.zeros((BL, H), jnp.float32)
    for t in range(T):
        xt = seq_ref[t]
        gates = (lax.dot_general(xt, wih_ref[...], dn,
                                 preferred_element_type=jnp.float32)
                 + lax.dot_general(h, whh_ref[...], dn,
                                   preferred_element_type=jnp.float32)
                 + b)
        i = jax.nn.sigmoid(gates[:, 0:H])
        f = jax.nn.sigmoid(gates[:, H:2 * H])
        g = jnp.tanh(gates[:, 2 * H:3 * H])
        o = jax.nn.sigmoid(gates[:, 3 * H:4 * H])
        c = f * c + i * g
        h = o * jnp.tanh(c)
    out_ref[...] = (lax.dot_general(h, wp_ref[...], dn,
                                    preferred_element_type=jnp.float32)
                    + bp_ref[...])


_lstm_call = pl.pallas_call(
    _lstm_body,
    grid=(NBL,),
    in_specs=[
        pl.BlockSpec((T, BL, H), lambda n: (0, n, 0)),
        pl.BlockSpec((4 * H, H), lambda n: (0, 0)),
        pl.BlockSpec((4 * H, H), lambda n: (0, 0)),
        pl.BlockSpec((1, 4 * H), lambda n: (0, 0)),
        pl.BlockSpec((1, 4 * H), lambda n: (0, 0)),
        pl.BlockSpec((F_DIM, H), lambda n: (0, 0)),
        pl.BlockSpec((1, F_DIM), lambda n: (0, 0)),
    ],
    out_specs=pl.BlockSpec((BL, F_DIM), lambda n: (n, 0)),
    out_shape=jax.ShapeDtypeStruct((N, F_DIM), jnp.float32),
)


def kernel(x_sequence, W_s1, b_s1, W_d1, b_d1, W_s2, b_s2, W_d2, b_d2,
           W_ih, W_hh, b_ih, b_hh, W_p, b_p, edge_index_sequence):
    ei = edge_index_sequence

    ones_chb = jnp.ones((CHB,), jnp.float32)
    zeros_hist = jnp.zeros((N // 2,), jnp.float32)
    zeros_rows = jnp.zeros((ROWS_PT, F_DIM), jnp.float32)

    deg_in, deg_out = _deg_call(ei, ones_chb, zeros_hist)
    deg_in = deg_in.reshape(T, NB, BN, 1)
    deg_out = deg_out.reshape(T, NB, BN, 1)

    xcat = _scale_call(x_sequence, deg_in, deg_out)
    u1 = _agg_call(xcat.reshape(PAIRS, N, F_DIM), ei, zeros_rows)
    h1cat = _combine1_call(u1, deg_in, deg_out, W_s1, b_s1.reshape(1, H),
                           W_d1, b_d1.reshape(1, H))
    u2 = _agg_call(h1cat.reshape(PAIRS, N, F_DIM), ei, zeros_rows)
    h2 = _combine2_call(u2, deg_in, deg_out, W_s2, b_s2.reshape(1, H),
                        W_d2, b_d2.reshape(1, H))
    out = _lstm_call(h2, W_ih, W_hh, b_ih.reshape(1, 4 * H),
                     b_hh.reshape(1, 4 * H), W_p, b_p.reshape(1, F_DIM))
    return out


# speedup vs baseline: 1.1271x; 1.0096x over previous
import functools

import jax
import jax.numpy as jnp
from jax import lax
from jax.experimental import pallas as pl
from jax.experimental.pallas import tpu as pltpu
from jax.experimental.pallas import tpu_sc as plsc

N = 10000
T = 8
F_DIM = 128
H = 128
E = 320000
ALPHA = 0.5

NC = 2
NS = 16
EPT = E // NS
CHB = 160
NCH = EPT // CHB
PAIRS = T * 2
PPC = PAIRS // NC
ROWS_PT = N // NS

_mesh = plsc.VectorSubcoreMesh(core_axis_name="c", subcore_axis_name="s")


def _deg_body(ei_hbm, ones_hbm, zeros_hbm, din_hbm, dout_hbm, idx_v, ones_v,
              bounce_v, sem, hist_sp):
    c = lax.axis_index("c")
    s = lax.axis_index("s")

    pltpu.sync_copy(ones_hbm, ones_v)
    pltpu.sync_copy(zeros_hbm, bounce_v)
    row = c * PPC + s // 2
    col = (s % 2) * (N // 2)
    pltpu.sync_copy(bounce_v, hist_sp.at[row, pl.ds(col, N // 2)])
    plsc.subcore_barrier()
    for tt in range(T // NC):
        for dd in range(2):
            t = c * (T // NC) + tt
            tp = t * 2 + dd
            pltpu.sync_copy(ei_hbm.at[t, 1 - dd, pl.ds(s * EPT, EPT)], idx_v)

            @pl.loop(0, NCH)
            def _(ch):
                pltpu.sync_copy(ones_v,
                                hist_sp.at[tp].at[idx_v.at[pl.ds(ch * CHB, CHB)]],
                                add=True)

    plsc.subcore_barrier()
    pltpu.sync_copy(hist_sp.at[row, pl.ds(col, N // 2)], bounce_v)
    t_row = row // 2

    @pl.when(row % 2 == 0)
    def _():
        pltpu.sync_copy(bounce_v, din_hbm.at[t_row, pl.ds(col, N // 2)])

    @pl.when(row % 2 == 1)
    def _():
        pltpu.sync_copy(bounce_v, dout_hbm.at[t_row, pl.ds(col, N // 2)])


_deg_call = pl.kernel(
    _deg_body,
    out_type=[jax.ShapeDtypeStruct((T, N), jnp.float32),
              jax.ShapeDtypeStruct((T, N), jnp.float32)],
    mesh=_mesh,
    compiler_params=pltpu.CompilerParams(use_tc_tiling_on_sc=False),
    scratch_types=[
        pltpu.VMEM((EPT,), jnp.int32),
        pltpu.VMEM((CHB,), jnp.float32),
        pltpu.VMEM((N // 2,), jnp.float32),
        pltpu.SemaphoreType.DMA,
        pltpu.VMEM_SHARED((PAIRS, N), jnp.float32),
    ],
)


NBLK = 5
NCHB = NCH // NBLK


def _agg_body(table_hbm, ei_hbm, zrows_hbm, out_hbm,
              gidx_v, sidx_v, rows_a, rows_b, sem_ga, sem_gb, sem_sa, sem_sb,
              sem_r, acc_sp):
    c = lax.axis_index("c")
    s = lax.axis_index("s")

    def wait_g(buf, sem):
        pltpu.make_async_copy(table_hbm.at[0, pl.ds(0, CHB)], buf, sem).wait()

    def sc(ch, buf, sem):
        pltpu.async_copy(buf, acc_sp.at[sidx_v.at[pl.ds(ch * CHB, CHB)]], sem,
                         add=True)

    def wait_s(buf, sem):
        pltpu.make_async_copy(buf, acc_sp.at[sidx_v.at[pl.ds(0, CHB)]], sem).wait()

    my_rows = pl.ds(s * ROWS_PT, ROWS_PT)

    def wait_readout():
        pltpu.make_async_copy(acc_sp.at[my_rows],
                              out_hbm.at[0, 0, pl.ds(0, ROWS_PT)], sem_r).wait()

    first = True
    for tt in range(T // NC):
        for dd in range(2):
            t = c * (T // NC) + tt
            tp = t * 2 + dd
            tbl = table_hbm.at[tp]

            def gg(ch, buf, sem):
                pltpu.async_copy(tbl.at[gidx_v.at[pl.ds(ch * CHB, CHB)]], buf, sem)

            def load_idx(hh):
                blk = pl.ds(s * EPT + hh * NCHB * CHB, NCHB * CHB)
                pltpu.sync_copy(ei_hbm.at[t, 1 - dd, blk], gidx_v)
                pltpu.sync_copy(ei_hbm.at[t, dd, blk], sidx_v)

            load_idx(0)
            gg(0, rows_a, sem_ga)
            gg(1, rows_b, sem_gb)
            if not first:
                wait_readout()
            first = False
            pltpu.sync_copy(zrows_hbm, acc_sp.at[my_rows])
            plsc.subcore_barrier()
            for hh in range(NBLK):
                if hh > 0:
                    load_idx(hh)
                    gg(0, rows_a, sem_ga)
                    gg(1, rows_b, sem_gb)
                wait_g(rows_a, sem_ga)
                sc(0, rows_a, sem_sa)

                @pl.loop(0, NCHB // 2 - 1)
                def _(i):
                    ch = 2 * i
                    wait_s(rows_a, sem_sa)
                    gg(ch + 2, rows_a, sem_ga)
                    wait_g(rows_b, sem_gb)
                    sc(ch + 1, rows_b, sem_sb)
                    wait_s(rows_b, sem_sb)
                    gg(ch + 3, rows_b, sem_gb)
                    wait_g(rows_a, sem_ga)
                    sc(ch + 2, rows_a, sem_sa)

                wait_s(rows_a, sem_sa)
                gg(NCHB - 1, rows_a, sem_ga)
                wait_g(rows_b, sem_gb)
                sc(NCHB - 2, rows_b, sem_sb)
                wait_g(rows_a, sem_ga)
                sc(NCHB - 1, rows_a, sem_sa)
                wait_s(rows_b, sem_sb)
                wait_s(rows_a, sem_sa)

            plsc.subcore_barrier()
            pltpu.async_copy(acc_sp.at[my_rows],
                             out_hbm.at[t, dd, my_rows], sem_r)
    wait_readout()


_agg_call = pl.kernel(
    _agg_body,
    out_type=jax.ShapeDtypeStruct((T, 2, N, F_DIM), jnp.float32),
    mesh=_mesh,
    compiler_params=pltpu.CompilerParams(use_tc_tiling_on_sc=False),
    scratch_types=[
        pltpu.VMEM((NCHB * CHB,), jnp.int32),
        pltpu.VMEM((NCHB * CHB,), jnp.int32),
        pltpu.VMEM((CHB, F_DIM), jnp.float32),
        pltpu.VMEM((CHB, F_DIM), jnp.float32),
        pltpu.SemaphoreType.DMA,
        pltpu.SemaphoreType.DMA,
        pltpu.SemaphoreType.DMA,
        pltpu.SemaphoreType.DMA,
        pltpu.SemaphoreType.DMA,
        pltpu.VMEM_SHARED((N, F_DIM), jnp.float32),
    ],
)


BN = 2000
NB = N // BN


def _inv_sqrt(d):
    return jnp.where(d > 0, lax.rsqrt(jnp.maximum(d, 1e-12)), 0.0)


def _scale_body(x_ref, di_ref, do_ref, out_ref):
    x = x_ref[0]
    inv_i = _inv_sqrt(di_ref[0, 0])
    inv_o = _inv_sqrt(do_ref[0, 0])
    out_ref[0, 0] = x * inv_i
    out_ref[0, 1] = x * inv_o


_scale_call = pl.pallas_call(
    _scale_body,
    grid=(T, NB),
    in_specs=[
        pl.BlockSpec((1, BN, F_DIM), lambda t, n: (t, n, 0)),
        pl.BlockSpec((1, 1, BN, 1), lambda t, n: (t, n, 0, 0)),
        pl.BlockSpec((1, 1, BN, 1), lambda t, n: (t, n, 0, 0)),
    ],
    out_specs=pl.BlockSpec((1, 2, BN, F_DIM), lambda t, n: (t, 0, n, 0)),
    out_shape=jax.ShapeDtypeStruct((T, 2, N, F_DIM), jnp.float32),
)


def _combine_body(u_ref, di_ref, do_ref, ws_ref, bs_ref, wd_ref, bd_ref,
                  out_ref, *, relu_and_scale):
    inv_i = _inv_sqrt(di_ref[0, 0])
    inv_o = _inv_sqrt(do_ref[0, 0])
    agg_fwd = u_ref[0, 0] * inv_o
    agg_rev = u_ref[0, 1] * inv_i
    dn = (((1,), (1,)), ((), ()))
    h = (ALPHA * (lax.dot_general(agg_fwd, ws_ref[...], dn,
                                  preferred_element_type=jnp.float32)
                  + bs_ref[...])
         + (1.0 - ALPHA) * (lax.dot_general(agg_rev, wd_ref[...], dn,
                                            preferred_element_type=jnp.float32)
                            + bd_ref[...]))
    if relu_and_scale:
        h = jnp.maximum(h, 0.0)
        out_ref[0, 0] = h * inv_i
        out_ref[0, 1] = h * inv_o
    else:
        out_ref[0] = h


def _make_combine(relu_and_scale):
    if relu_and_scale:
        out_specs = pl.BlockSpec((1, 2, BN, F_DIM), lambda t, n: (t, 0, n, 0))
        out_shape = jax.ShapeDtypeStruct((T, 2, N, F_DIM), jnp.float32)
    else:
        out_specs = pl.BlockSpec((1, BN, F_DIM), lambda t, n: (t, n, 0))
        out_shape = jax.ShapeDtypeStruct((T, N, F_DIM), jnp.float32)
    return pl.pallas_call(
        functools.partial(_combine_body, relu_and_scale=relu_and_scale),
        grid=(T, NB),
        in_specs=[
            pl.BlockSpec((1, 2, BN, F_DIM), lambda t, n: (t, 0, n, 0)),
            pl.BlockSpec((1, 1, BN, 1), lambda t, n: (t, n, 0, 0)),
            pl.BlockSpec((1, 1, BN, 1), lambda t, n: (t, n, 0, 0)),
            pl.BlockSpec((H, F_DIM), lambda t, n: (0, 0)),
            pl.BlockSpec((1, H), lambda t, n: (0, 0)),
            pl.BlockSpec((H, F_DIM), lambda t, n: (0, 0)),
            pl.BlockSpec((1, H), lambda t, n: (0, 0)),
        ],
        out_specs=out_specs,
        out_shape=out_shape,
    )


_combine1_call = _make_combine(True)
_combine2_call = _make_combine(False)

BL = 2000
NBL = N // BL


def _lstm_body(seq_ref, wih_ref, whh_ref, bih_ref, bhh_ref, wp_ref, bp_ref,
               out_ref):
    dn = (((1,), (1,)), ((), ()))
    b = bih_ref[...] + bhh_ref[...]
    h = jnp.zeros((BL, H), jnp.float32)
    c = jnp.zeros((BL, H), jnp.float32)
    for t in range(T):
        xt = seq_ref[t]
        gates = (lax.dot_general(xt, wih_ref[...], dn,
                                 preferred_element_type=jnp.float32)
                 + lax.dot_general(h, whh_ref[...], dn,
                                   preferred_element_type=jnp.float32)
                 + b)
        i = jax.nn.sigmoid(gates[:, 0:H])
        f = jax.nn.sigmoid(gates[:, H:2 * H])
        g = jnp.tanh(gates[:, 2 * H:3 * H])
        o = jax.nn.sigmoid(gates[:, 3 * H:4 * H])
        c = f * c + i * g
        h = o * jnp.tanh(c)
    out_ref[...] = (lax.dot_general(h, wp_ref[...], dn,
                                    preferred_element_type=jnp.float32)
                    + bp_ref[...])


_lstm_call = pl.pallas_call(
    _lstm_body,
    grid=(NBL,),
    in_specs=[
        pl.BlockSpec((T, BL, H), lambda n: (0, n, 0)),
        pl.BlockSpec((4 * H, H), lambda n: (0, 0)),
        pl.BlockSpec((4 * H, H), lambda n: (0, 0)),
        pl.BlockSpec((1, 4 * H), lambda n: (0, 0)),
        pl.BlockSpec((1, 4 * H), lambda n: (0, 0)),
        pl.BlockSpec((F_DIM, H), lambda n: (0, 0)),
        pl.BlockSpec((1, F_DIM), lambda n: (0, 0)),
    ],
    out_specs=pl.BlockSpec((BL, F_DIM), lambda n: (n, 0)),
    out_shape=jax.ShapeDtypeStruct((N, F_DIM), jnp.float32),
)


def kernel(x_sequence, W_s1, b_s1, W_d1, b_d1, W_s2, b_s2, W_d2, b_d2,
           W_ih, W_hh, b_ih, b_hh, W_p, b_p, edge_index_sequence):
    ei = edge_index_sequence

    ones_chb = jnp.ones((CHB,), jnp.float32)
    zeros_hist = jnp.zeros((N // 2,), jnp.float32)
    zeros_rows = jnp.zeros((ROWS_PT, F_DIM), jnp.float32)

    deg_in, deg_out = _deg_call(ei, ones_chb, zeros_hist)
    deg_in = deg_in.reshape(T, NB, BN, 1)
    deg_out = deg_out.reshape(T, NB, BN, 1)

    xcat = _scale_call(x_sequence, deg_in, deg_out)
    u1 = _agg_call(xcat.reshape(PAIRS, N, F_DIM), ei, zeros_rows)
    h1cat = _combine1_call(u1, deg_in, deg_out, W_s1, b_s1.reshape(1, H),
                           W_d1, b_d1.reshape(1, H))
    u2 = _agg_call(h1cat.reshape(PAIRS, N, F_DIM), ei, zeros_rows)
    h2 = _combine2_call(u2, deg_in, deg_out, W_s2, b_s2.reshape(1, H),
                        W_d2, b_d2.reshape(1, H))
    out = _lstm_call(h2, W_ih, W_hh, b_ih.reshape(1, 4 * H),
                     b_hh.reshape(1, 4 * H), W_p, b_p.reshape(1, F_DIM))
    return out
